# baseline (speedup 1.0000x reference)
import functools

import jax
import jax.numpy as jnp
from jax import lax
from jax.experimental import pallas as pl
from jax.experimental.pallas import tpu as pltpu
from jax.experimental.pallas import tpu_sc as plsc

F32 = jnp.float32

NC = 2
NS = 16
LANES = 16

GK = 128


def _silu(x):
    return x * (1.0 / (1.0 + jnp.exp(-x)))


def _edge_body(hs_ref, hd_ref, e_ref, wa_ref, wb_ref, ws_ref, b1_ref,
               w2_ref, e2_ref, mf_ref):
    e0 = e_ref[:, 0, :]
    e1 = e_ref[:, 1, :]
    es = (e0 + e1) * 0.5
    s = jnp.dot(es, ws_ref[...], preferred_element_type=F32) + b1_ref[...]
    hs = hs_ref[0]
    hd = hd_ref[0]
    x1 = (jnp.dot(hs, wa_ref[...], preferred_element_type=F32)
          + jnp.dot(hd, wb_ref[...], preferred_element_type=F32) + s)
    x2 = (jnp.dot(hd, wa_ref[...], preferred_element_type=F32)
          + jnp.dot(hs, wb_ref[...], preferred_element_type=F32) + s)
    d = _silu(x1) - _silu(x2)
    mf = jnp.dot(d, w2_ref[...], preferred_element_type=F32)
    mf_ref[...] = mf
    e2_ref[:, 0, :] = e0 + mf
    e2_ref[:, 1, :] = e1 - mf


def _node_body(h_ref, agg_ref, wnh_ref, wna_ref, b1_ref, w2_ref, b2_ref,
               out_ref):
    agg = agg_ref[0] - agg_ref[1]
    t = (jnp.dot(h_ref[...], wnh_ref[...], preferred_element_type=F32)
         + jnp.dot(agg, wna_ref[...], preferred_element_type=F32)
         + b1_ref[...])
    t = _silu(t)
    out_ref[...] = (h_ref[...]
                    + jnp.dot(t, w2_ref[...], preferred_element_type=F32)
                    + b2_ref[...])


def kernel(h, e, ei, W_e1, b_e1, W_e2, b_e2, W_n1, b_n1, W_n2, b_n2):
    B, N, H = h.shape
    E = e.shape[1]
    P = E // 2

    h2 = h.reshape(N, H)
    e3 = e.reshape(P, 2, H)

    src_e = ei[0, 0::2].reshape(1, P)
    dst_e = ei[1, 0::2].reshape(1, P)
    dst_o = ei[1, 1::2].reshape(1, P)

    wa = W_e1[:H]
    wb = W_e1[H:2 * H]
    ws = W_e1[2 * H:]
    b1 = b_e1.reshape(1, H)
    wnh = W_n1[:H]
    wna = W_n1[H:]
    bn1 = b_n1.reshape(1, H)
    bn2 = b_n2.reshape(1, H)

    mesh = plsc.VectorSubcoreMesh(core_axis_name="c", subcore_axis_name="s",
                                  num_cores=NC, num_subcores=NS)

    idx_sd = jnp.stack([src_e, dst_e])

    GB = 2 * GK

    @functools.partial(
        pl.kernel,
        out_type=jax.ShapeDtypeStruct((2, P, H), F32),
        mesh=mesh,
        scratch_types=[pltpu.SemaphoreType.DMA],
    )
    def _gather_k(h_hbm, idx_hbm, out_hbm, sem):
        def body(i_v, o_v):
            cp0 = pltpu.async_copy(
                h_hbm.at[i_v.at[0, 0, pl.ds(0, GK)]],
                o_v.at[0, pl.ds(0, GK)], sem)
            cp1 = pltpu.async_copy(
                h_hbm.at[i_v.at[0, 0, pl.ds(GK, GK)]],
                o_v.at[0, pl.ds(GK, GK)], sem)
            cp0.wait()
            cp1.wait()

        pltpu.emit_pipeline(
            body,
            grid=(2, P // GB),
            in_specs=[pl.BlockSpec((1, 1, GB), lambda j, i: (j, 0, i))],
            out_specs=[pl.BlockSpec((1, GB, H), lambda j, i: (j, i, 0))],
            core_axis_name=("c", "s"),
            dimension_semantics=(pltpu.PARALLEL, pltpu.PARALLEL),
        )(idx_hbm, out_hbm)

    h_sd = _gather_k(h2, idx_sd)

    BP = 5000
    e2, mf = pl.pallas_call(
        _edge_body,
        grid=(P // BP,),
        in_specs=[
            pl.BlockSpec((1, BP, H), lambda i: (0, i, 0)),
            pl.BlockSpec((1, BP, H), lambda i: (1, i, 0)),
            pl.BlockSpec((BP, 2, H), lambda i: (i, 0, 0)),
            pl.BlockSpec((H, H), lambda i: (0, 0)),
            pl.BlockSpec((H, H), lambda i: (0, 0)),
            pl.BlockSpec((H, H), lambda i: (0, 0)),
            pl.BlockSpec((1, H), lambda i: (0, 0)),
            pl.BlockSpec((H, H), lambda i: (0, 0)),
        ],
        out_specs=[
            pl.BlockSpec((BP, 2, H), lambda i: (i, 0, 0)),
            pl.BlockSpec((BP, H), lambda i: (i, 0)),
        ],
        out_shape=[
            jax.ShapeDtypeStruct((P, 2, H), F32),
            jax.ShapeDtypeStruct((P, H), F32),
        ],
    )(h_sd, h_sd, e3, wa, wb, ws, b1, W_e2)

    CW = 80
    NCH = N // CW
    KMAX = -(-NCH // NS)

    @functools.partial(
        pl.kernel,
        out_type=jax.ShapeDtypeStruct((NC, N, H), F32),
        mesh=mesh,
        scratch_types=[
            pltpu.VMEM((CW, H), F32),
            pltpu.VMEM_SHARED((N, H), F32),
            pltpu.SemaphoreType.DMA,
        ],
    )
    def _scatter_k(m_hbm, ie_hbm, io_hbm, out_hbm, z_v, agg_sh, sem):
        cid = lax.axis_index("c")
        sid = lax.axis_index("s")

        @pl.loop(0, CW)
        def _zrow(r):
            @pl.loop(0, H, step=LANES)
            def _zcol(cc):
                z_v[r, pl.ds(cc, LANES)] = jnp.zeros((LANES,), F32)

        @pl.loop(0, KMAX)
        def _zcp(k):
            c = sid + NS * k

            @pl.when(c < NCH)
            def _():
                pltpu.sync_copy(z_v, agg_sh.at[pl.ds(c * CW, CW)])

        plsc.subcore_barrier()

        def body(m_v, ie_v, io_v):
            @pl.when(cid == 0)
            def _():
                pltpu.sync_copy(m_v, agg_sh.at[ie_v.at[0, 0]], add=True)

            @pl.when(cid == 1)
            def _():
                pltpu.sync_copy(m_v, agg_sh.at[io_v.at[0, 0]], add=True)

        pltpu.emit_pipeline(
            body,
            grid=(P // GK,),
            in_specs=[pl.BlockSpec((GK, H), lambda i: (i, 0)),
                      pl.BlockSpec((1, 1, GK), lambda i: (i, 0, 0)),
                      pl.BlockSpec((1, 1, GK), lambda i: (i, 0, 0))],
            out_specs=[],
            core_axis_name="s",
            dimension_semantics=(pltpu.PARALLEL,),
        )(m_hbm, ie_hbm, io_hbm)

        plsc.subcore_barrier()

        @pl.loop(0, KMAX)
        def _wcp(k):
            c = sid + NS * k

            @pl.when(c < NCH)
            def _():
                pltpu.sync_copy(agg_sh.at[pl.ds(c * CW, CW)],
                                out_hbm.at[cid, pl.ds(c * CW, CW)])

    agg2 = _scatter_k(mf,
                      dst_e.reshape(P // GK, 1, GK),
                      dst_o.reshape(P // GK, 1, GK))

    hn = pl.pallas_call(
        _node_body,
        out_shape=jax.ShapeDtypeStruct((N, H), F32),
    )(h2, agg2, wnh, wna, bn1, W_n2, bn2)

    return hn.reshape(B, N, H), e2.reshape(B, E, H)


# speedup vs baseline: 1.0349x; 1.0349x over previous
import functools

import jax
import jax.numpy as jnp
from jax import lax
from jax.experimental import pallas as pl
from jax.experimental.pallas import tpu as pltpu
from jax.experimental.pallas import tpu_sc as plsc

F32 = jnp.float32

NC = 2
NS = 16
LANES = 16

GK = 128


def _silu(x):
    return x * (1.0 / (1.0 + jnp.exp(-x)))


def _edge_body(hs_ref, hd_ref, e_ref, wa_ref, wb_ref, ws_ref, b1_ref,
               w2_ref, e2_ref, mf_ref):
    e0 = e_ref[:, 0, :]
    e1 = e_ref[:, 1, :]
    es = (e0 + e1) * 0.5
    s = jnp.dot(es, ws_ref[...], preferred_element_type=F32) + b1_ref[...]
    hs = hs_ref[0]
    hd = hd_ref[0]
    x1 = (jnp.dot(hs, wa_ref[...], preferred_element_type=F32)
          + jnp.dot(hd, wb_ref[...], preferred_element_type=F32) + s)
    x2 = (jnp.dot(hd, wa_ref[...], preferred_element_type=F32)
          + jnp.dot(hs, wb_ref[...], preferred_element_type=F32) + s)
    d = _silu(x1) - _silu(x2)
    mf = jnp.dot(d, w2_ref[...], preferred_element_type=F32)
    mf_ref[...] = mf
    e2_ref[:, 0, :] = e0 + mf
    e2_ref[:, 1, :] = e1 - mf


def _node_body(h_ref, agg_ref, wnh_ref, wna_ref, b1_ref, w2_ref, b2_ref,
               out_ref):
    agg = agg_ref[0] - agg_ref[1]
    t = (jnp.dot(h_ref[...], wnh_ref[...], preferred_element_type=F32)
         + jnp.dot(agg, wna_ref[...], preferred_element_type=F32)
         + b1_ref[...])
    t = _silu(t)
    out_ref[...] = (h_ref[...]
                    + jnp.dot(t, w2_ref[...], preferred_element_type=F32)
                    + b2_ref[...])


def kernel(h, e, ei, W_e1, b_e1, W_e2, b_e2, W_n1, b_n1, W_n2, b_n2):
    B, N, H = h.shape
    E = e.shape[1]
    P = E // 2

    h2 = h.reshape(N, H)
    e3 = e.reshape(P, 2, H)

    src_e = ei[0, 0::2].reshape(1, P)
    dst_e = ei[1, 0::2].reshape(1, P)
    dst_o = ei[1, 1::2].reshape(1, P)

    wa = W_e1[:H]
    wb = W_e1[H:2 * H]
    ws = W_e1[2 * H:]
    b1 = b_e1.reshape(1, H)
    wnh = W_n1[:H]
    wna = W_n1[H:]
    bn1 = b_n1.reshape(1, H)
    bn2 = b_n2.reshape(1, H)

    mesh = plsc.VectorSubcoreMesh(core_axis_name="c", subcore_axis_name="s",
                                  num_cores=NC, num_subcores=NS)

    idx_sd = jnp.stack([src_e, dst_e])

    GB = 2 * GK

    @functools.partial(
        pl.kernel,
        out_type=jax.ShapeDtypeStruct((2, P, H), F32),
        mesh=mesh,
        scratch_types=[pltpu.SemaphoreType.DMA],
    )
    def _gather_k(h_hbm, idx_hbm, out_hbm, sem):
        def body(i_v, o_v):
            cp0 = pltpu.async_copy(
                h_hbm.at[i_v.at[0, 0, pl.ds(0, GK)]],
                o_v.at[0, pl.ds(0, GK)], sem)
            cp1 = pltpu.async_copy(
                h_hbm.at[i_v.at[0, 0, pl.ds(GK, GK)]],
                o_v.at[0, pl.ds(GK, GK)], sem)
            cp0.wait()
            cp1.wait()

        pltpu.emit_pipeline(
            body,
            grid=(2, P // GB),
            in_specs=[pl.BlockSpec((1, 1, GB), lambda j, i: (j, 0, i))],
            out_specs=[pl.BlockSpec((1, GB, H), lambda j, i: (j, i, 0))],
            core_axis_name=("c", "s"),
            dimension_semantics=(pltpu.PARALLEL, pltpu.PARALLEL),
        )(idx_hbm, out_hbm)

    h_sd = _gather_k(h2, idx_sd)

    BP = 4000
    e2, mf = pl.pallas_call(
        _edge_body,
        grid=(P // BP,),
        in_specs=[
            pl.BlockSpec((1, BP, H), lambda i: (0, i, 0)),
            pl.BlockSpec((1, BP, H), lambda i: (1, i, 0)),
            pl.BlockSpec((BP, 2, H), lambda i: (i, 0, 0)),
            pl.BlockSpec((H, H), lambda i: (0, 0)),
            pl.BlockSpec((H, H), lambda i: (0, 0)),
            pl.BlockSpec((H, H), lambda i: (0, 0)),
            pl.BlockSpec((1, H), lambda i: (0, 0)),
            pl.BlockSpec((H, H), lambda i: (0, 0)),
        ],
        out_specs=[
            pl.BlockSpec((BP, 2, H), lambda i: (i, 0, 0)),
            pl.BlockSpec((BP, H), lambda i: (i, 0)),
        ],
        out_shape=[
            jax.ShapeDtypeStruct((P, 2, H), F32),
            jax.ShapeDtypeStruct((P, H), F32),
        ],
    )(h_sd, h_sd, e3, wa, wb, ws, b1, W_e2)

    CW = 80
    NCH = N // CW
    KMAX = -(-NCH // NS)

    @functools.partial(
        pl.kernel,
        out_type=jax.ShapeDtypeStruct((NC, N, H), F32),
        mesh=mesh,
        scratch_types=[
            pltpu.VMEM((CW, H), F32),
            pltpu.VMEM_SHARED((N, H), F32),
            pltpu.SemaphoreType.DMA,
        ],
    )
    def _scatter_k(m_hbm, ie_hbm, io_hbm, out_hbm, z_v, agg_sh, sem):
        cid = lax.axis_index("c")
        sid = lax.axis_index("s")

        @pl.loop(0, CW)
        def _zrow(r):
            @pl.loop(0, H, step=LANES)
            def _zcol(cc):
                z_v[r, pl.ds(cc, LANES)] = jnp.zeros((LANES,), F32)

        @pl.loop(0, KMAX)
        def _zcp(k):
            c = sid + NS * k

            @pl.when(c < NCH)
            def _():
                pltpu.sync_copy(z_v, agg_sh.at[pl.ds(c * CW, CW)])

        plsc.subcore_barrier()

        def body(m_v, ie_v, io_v):
            @pl.when(cid == 0)
            def _():
                pltpu.sync_copy(m_v, agg_sh.at[ie_v.at[0, 0]], add=True)

            @pl.when(cid == 1)
            def _():
                pltpu.sync_copy(m_v, agg_sh.at[io_v.at[0, 0]], add=True)

        pltpu.emit_pipeline(
            body,
            grid=(P // GK,),
            in_specs=[pl.BlockSpec((GK, H), lambda i: (i, 0)),
                      pl.BlockSpec((1, 1, GK), lambda i: (i, 0, 0)),
                      pl.BlockSpec((1, 1, GK), lambda i: (i, 0, 0))],
            out_specs=[],
            core_axis_name="s",
            dimension_semantics=(pltpu.PARALLEL,),
        )(m_hbm, ie_hbm, io_hbm)

        plsc.subcore_barrier()

        @pl.loop(0, KMAX)
        def _wcp(k):
            c = sid + NS * k

            @pl.when(c < NCH)
            def _():
                pltpu.sync_copy(agg_sh.at[pl.ds(c * CW, CW)],
                                out_hbm.at[cid, pl.ds(c * CW, CW)])

    agg2 = _scatter_k(mf,
                      dst_e.reshape(P // GK, 1, GK),
                      dst_o.reshape(P // GK, 1, GK))

    hn = pl.pallas_call(
        _node_body,
        out_shape=jax.ShapeDtypeStruct((N, H), F32),
    )(h2, agg2, wnh, wna, bn1, W_n2, bn2)

    return hn.reshape(B, N, H), e2.reshape(B, E, H)
